# baseline (speedup 1.0000x reference)
import functools
import jax
import jax.numpy as jnp
from jax import lax
from jax.experimental import pallas as pl
from jax.experimental.pallas import tpu as pltpu

N = 50000
NNZ = 800000
EPS = 1e-5

_BE = 6400
_BN = 2000
_GE = NNZ // _BE
_GN = N // _BN


def _gram_body(f_ref, g_ref):
    i = pl.program_id(0)

    @pl.when(i == 0)
    def _():
        g_ref[...] = jnp.zeros_like(g_ref)

    f = f_ref[...]
    g_ref[...] += jax.lax.dot_general(f, f, (((0,), (0,)), ((), ())),
                                      preferred_element_type=jnp.float32)


def _gram(f8):
    return pl.pallas_call(
        _gram_body,
        grid=(_GE,),
        in_specs=[pl.BlockSpec((_BE, 8), lambda i: (i, 0))],
        out_specs=pl.BlockSpec((8, 8), lambda i: (0, 0)),
        out_shape=jax.ShapeDtypeStruct((8, 8), jnp.float32),
    )(f8)


def _vals1_body(f_ref, v_ref, lo_ref, hi_ref):
    f = f_ref[...]
    v = v_ref[...]
    o = jnp.maximum(jax.lax.dot_general(f, v, (((1,), (0,)), ((), ())),
                                        preferred_element_type=jnp.float32), 0.0)
    lo_ref[...] = o[:, :32]
    hi_ref[...] = o[:, 32:]


def _vals1(f8, v1):
    return pl.pallas_call(
        _vals1_body,
        grid=(_GE,),
        in_specs=[pl.BlockSpec((_BE, 8), lambda i: (i, 0)),
                  pl.BlockSpec((8, 64), lambda i: (0, 0))],
        out_specs=[pl.BlockSpec((_BE, 32), lambda i: (i, 0)),
                   pl.BlockSpec((_BE, 32), lambda i: (i, 0))],
        out_shape=[jax.ShapeDtypeStruct((NNZ, 32), jnp.float32),
                   jax.ShapeDtypeStruct((NNZ, 32), jnp.float32)],
    )(f8, v1)


def _tabs_body(rs_ref, cs_ref, rc_ref, cc_ref, fi_ref, v1_ref, u1_ref, u2_ref,
               u5_ref, rtab_ref, ctab_ref, gsum_ref):
    i = pl.program_id(0)
    rc = jnp.maximum(rc_ref[...], 1.0)
    cc = jnp.maximum(cc_ref[...], 1.0)
    rs = rs_ref[...]
    rm2 = rs / rc
    cm2 = cs_ref[...] / cc
    diag2 = jnp.maximum(
        jax.lax.dot_general(fi_ref[...], v1_ref[...], (((1,), (0,)), ((), ())),
                            preferred_element_type=jnp.float32), 0.0)
    rtab_ref[...] = (
        jax.lax.dot_general(rm2, u1_ref[...], (((1,), (0,)), ((), ())),
                            preferred_element_type=jnp.float32)
        + jax.lax.dot_general(diag2, u5_ref[...], (((1,), (0,)), ((), ())),
                              preferred_element_type=jnp.float32))
    ctab_ref[...] = jax.lax.dot_general(cm2, u2_ref[...], (((1,), (0,)), ((), ())),
                                        preferred_element_type=jnp.float32)

    @pl.when(i == 0)
    def _():
        gsum_ref[...] = jnp.zeros_like(gsum_ref)

    gsum_ref[...] += jnp.sum(rs, axis=0, keepdims=True)


def _tabs(rowsum2, colsum2, rc, cc, fi8, v1, u1, u2, u5):
    return pl.pallas_call(
        _tabs_body,
        grid=(_GN,),
        in_specs=[pl.BlockSpec((_BN, 64), lambda i: (i, 0)),
                  pl.BlockSpec((_BN, 64), lambda i: (i, 0)),
                  pl.BlockSpec((_BN, 1), lambda i: (i, 0)),
                  pl.BlockSpec((_BN, 1), lambda i: (i, 0)),
                  pl.BlockSpec((_BN, 8), lambda i: (i, 0)),
                  pl.BlockSpec((8, 64), lambda i: (0, 0)),
                  pl.BlockSpec((64, 64), lambda i: (0, 0)),
                  pl.BlockSpec((64, 64), lambda i: (0, 0)),
                  pl.BlockSpec((64, 64), lambda i: (0, 0))],
        out_specs=[pl.BlockSpec((_BN, 64), lambda i: (i, 0)),
                   pl.BlockSpec((_BN, 64), lambda i: (i, 0)),
                   pl.BlockSpec((1, 64), lambda i: (0, 0))],
        out_shape=[jax.ShapeDtypeStruct((N, 64), jnp.float32),
                   jax.ShapeDtypeStruct((N, 64), jnp.float32),
                   jax.ShapeDtypeStruct((1, 64), jnp.float32)],
    )(rowsum2, colsum2, rc, cc, fi8, v1, u1, u2, u5)


def _out2_body(f_ref, ft_ref, re_ref, ce_ref, v1_ref, u0_ref, u4_ref, k2_ref,
               o_ref, s_ref):
    i = pl.program_id(0)
    v1 = v1_ref[...]
    v1a = jnp.maximum(
        jax.lax.dot_general(f_ref[...], v1, (((1,), (0,)), ((), ())),
                            preferred_element_type=jnp.float32), 0.0)
    t2 = jnp.maximum(
        jax.lax.dot_general(ft_ref[...], v1, (((1,), (0,)), ((), ())),
                            preferred_element_type=jnp.float32), 0.0)
    o = (jax.lax.dot_general(v1a, u0_ref[...], (((1,), (0,)), ((), ())),
                             preferred_element_type=jnp.float32)
         + jax.lax.dot_general(t2, u4_ref[...], (((1,), (0,)), ((), ())),
                               preferred_element_type=jnp.float32)
         + re_ref[...].astype(jnp.float32) + ce_ref[...].astype(jnp.float32)
         + k2_ref[...])
    o_ref[...] = o

    @pl.when(i == 0)
    def _():
        s_ref[...] = jnp.zeros_like(s_ref)

    s_ref[0:1, :] += jnp.sum(o, axis=0, keepdims=True)
    s_ref[1:2, :] += jnp.sum(o * o, axis=0, keepdims=True)


def _out2(f8, ft8, re, ce, v1, u0, u4, k2):
    return pl.pallas_call(
        _out2_body,
        grid=(_GE,),
        in_specs=[pl.BlockSpec((_BE, 8), lambda i: (i, 0)),
                  pl.BlockSpec((_BE, 8), lambda i: (i, 0)),
                  pl.BlockSpec((_BE, 64), lambda i: (i, 0)),
                  pl.BlockSpec((_BE, 64), lambda i: (i, 0)),
                  pl.BlockSpec((8, 64), lambda i: (0, 0)),
                  pl.BlockSpec((64, 64), lambda i: (0, 0)),
                  pl.BlockSpec((64, 64), lambda i: (0, 0)),
                  pl.BlockSpec((1, 64), lambda i: (0, 0))],
        out_specs=[pl.BlockSpec((_BE, 64), lambda i: (i, 0)),
                   pl.BlockSpec((2, 64), lambda i: (0, 0))],
        out_shape=[jax.ShapeDtypeStruct((NNZ, 64), jnp.float32),
                   jax.ShapeDtypeStruct((2, 64), jnp.float32)],
    )(f8, ft8, re, ce, v1, u0, u4, k2)


def _vals2_body(o_ref, m_ref, lo_ref, hi_ref):
    o = o_ref[...]
    v = jnp.maximum((o - m_ref[0:1, :]) * m_ref[1:2, :], 0.0)
    lo_ref[...] = v[:, :32]
    hi_ref[...] = v[:, 32:]


def _vals2(out2, musig):
    return pl.pallas_call(
        _vals2_body,
        grid=(_GE,),
        in_specs=[pl.BlockSpec((_BE, 64), lambda i: (i, 0)),
                  pl.BlockSpec((2, 64), lambda i: (0, 0))],
        out_specs=[pl.BlockSpec((_BE, 32), lambda i: (i, 0)),
                   pl.BlockSpec((_BE, 32), lambda i: (i, 0))],
        out_shape=[jax.ShapeDtypeStruct((NNZ, 32), jnp.float32),
                   jax.ShapeDtypeStruct((NNZ, 32), jnp.float32)],
    )(out2, musig)


def _emb_body(p_ref, rc_ref, w_ref, b_ref, o_ref):
    ent = p_ref[...] / jnp.maximum(rc_ref[...], 1.0)
    o_ref[...] = (jax.lax.dot_general(ent, w_ref[...], (((1,), (0,)), ((), ())),
                                      preferred_element_type=jnp.float32)
                  + b_ref[...])


def _emb(pooled, rc, w_pool, b_pool):
    return pl.pallas_call(
        _emb_body,
        grid=(_GN,),
        in_specs=[pl.BlockSpec((_BN, 64), lambda i: (i, 0)),
                  pl.BlockSpec((_BN, 1), lambda i: (i, 0)),
                  pl.BlockSpec((64, 50), lambda i: (0, 0)),
                  pl.BlockSpec((1, 50), lambda i: (0, 0))],
        out_specs=pl.BlockSpec((_BN, 50), lambda i: (i, 0)),
        out_shape=jax.ShapeDtypeStruct((N, 50), jnp.float32),
    )(pooled, rc, w_pool, b_pool[None, :])


def kernel(data_values, data_indices, idx_identity, idx_transpose, W_in, b_in,
           Ws, bs, W_pool, b_pool):
    d = data_values[:, 0]
    row = data_indices[0]
    col = data_indices[1]
    ones = jnp.ones((NNZ,), jnp.float32)

    rs = jax.ops.segment_sum(d, row, num_segments=N)
    rc = jax.ops.segment_sum(ones, row, num_segments=N)
    cs = jax.ops.segment_sum(d, col, num_segments=N)
    cc = jax.ops.segment_sum(ones, col, num_segments=N)
    rmd = rs / jnp.maximum(rc, 1.0)
    mr = jnp.minimum(rc, 1.0)
    cmd = cs / jnp.maximum(cc, 1.0)
    mc = jnp.minimum(cc, 1.0)
    dt = d[idx_transpose]
    di = d[idx_identity]

    w = W_in[0]
    b = b_in
    W0, W1, W2, W3, W4, W5 = [Ws[0, i] for i in range(6)]
    gm = jnp.sum(rs) / NNZ
    k = gm * (w @ W3) + b @ (W0 + W3 + W4 + W5) + bs[0]
    V8 = jnp.stack([w @ W0, w @ W4, w @ W1, w @ W2, w @ W5, b @ W1, b @ W2, k],
                   axis=0)

    f8 = jnp.stack([d, dt, rmd[row], cmd[col], di[row], mr[row], mc[col], ones],
                   axis=1)

    G = _gram(f8)
    mA = G[7, :] / NNZ
    mu1 = mA @ V8
    Eo2 = jnp.einsum('ic,ij,jc->c', V8, G / NNZ, V8)
    var1 = Eo2 - mu1 * mu1
    sig1 = jnp.sqrt(var1 + EPS)
    V1 = (V8 - jnp.eye(8, dtype=jnp.float32)[:, 7:8] * mu1[None, :]) / sig1[None, :]

    v1lo, v1hi = _vals1(f8, V1)
    vals1 = jnp.concatenate([v1lo, v1hi], axis=1)
    rowsum2 = jax.ops.segment_sum(vals1, row, num_segments=N)
    colsum2 = jax.ops.segment_sum(vals1, col, num_segments=N)

    ft8 = f8[idx_transpose]
    fi8 = f8[idx_identity]

    U0, U1, U2, U3, U4, U5 = [Ws[1, i] for i in range(6)]
    rtab, ctab, gsum = _tabs(rowsum2, colsum2, rc[:, None], cc[:, None], fi8,
                             V1, U1, U2, U5)
    g2 = gsum[0] / NNZ
    k2 = (g2 @ U3 + bs[1])[None, :]

    re = rtab[row]
    ce = ctab[col]

    out2, s12 = _out2(f8, ft8, re, ce, V1, U0, U4, k2)
    mu2 = s12[0] / NNZ
    var2 = s12[1] / NNZ - mu2 * mu2
    musig = jnp.stack([mu2, 1.0 / jnp.sqrt(var2 + EPS)], axis=0)
    v2lo, v2hi = _vals2(out2, musig)
    vals2 = jnp.concatenate([v2lo, v2hi], axis=1)
    pooled = jax.ops.segment_sum(vals2, row, num_segments=N)
    return _emb(pooled, rc[:, None], W_pool, b_pool)


# speedup vs baseline: 1.1676x; 1.1676x over previous
import functools
import jax
import jax.numpy as jnp
from jax import lax
from jax.experimental import pallas as pl
from jax.experimental.pallas import tpu as pltpu
from jax.experimental.pallas import tpu_sc as plsc

N = 50000
NNZ = 800000
H = NNZ // 2
EPS = 1e-5

_BE = 6400
_B2 = 3200
_BN = 2000
_GE = NNZ // _BE
_G2 = H // _B2
_GN = N // _BN


def _gram_body(f_ref, g_ref):
    i = pl.program_id(0)

    @pl.when(i == 0)
    def _():
        g_ref[...] = jnp.zeros_like(g_ref)

    f = f_ref[...]
    g_ref[...] += jax.lax.dot_general(f, f, (((1,), (1,)), ((), ())),
                                      preferred_element_type=jnp.float32)


def _gram(f8cm):
    return pl.pallas_call(
        _gram_body,
        grid=(_GE,),
        in_specs=[pl.BlockSpec((8, _BE), lambda i: (0, i))],
        out_specs=pl.BlockSpec((8, 8), lambda i: (0, 0)),
        out_shape=jax.ShapeDtypeStruct((8, 8), jnp.float32),
    )(f8cm)


def _vals1_body(fl_ref, fh_ref, v_ref, o_ref):
    v = v_ref[...]
    o_ref[:, :64] = jnp.maximum(
        jax.lax.dot_general(fl_ref[...], v, (((0,), (0,)), ((), ())),
                            preferred_element_type=jnp.float32), 0.0)
    o_ref[:, 64:] = jnp.maximum(
        jax.lax.dot_general(fh_ref[...], v, (((0,), (0,)), ((), ())),
                            preferred_element_type=jnp.float32), 0.0)


def _vals1(f8cm, v1):
    return pl.pallas_call(
        _vals1_body,
        grid=(_G2,),
        in_specs=[pl.BlockSpec((8, _B2), lambda i: (0, i)),
                  pl.BlockSpec((8, _B2), lambda i: (0, i + _G2)),
                  pl.BlockSpec((8, 64), lambda i: (0, 0))],
        out_specs=pl.BlockSpec((_B2, 128), lambda i: (i, 0)),
        out_shape=jax.ShapeDtypeStruct((H, 128), jnp.float32),
    )(f8cm, f8cm, v1)


def _tabs_body(rs_ref, cs_ref, rc_ref, cc_ref, fi_ref, v1_ref, u1_ref, u2_ref,
               u5_ref, tab_ref, gsum_ref):
    i = pl.program_id(0)
    rc = jnp.maximum(rc_ref[...], 1.0)
    cc = jnp.maximum(cc_ref[...], 1.0)
    rs = rs_ref[...]
    rm2 = rs / rc
    cm2 = cs_ref[...] / cc
    diag2 = jnp.maximum(
        jax.lax.dot_general(fi_ref[...], v1_ref[...], (((1,), (0,)), ((), ())),
                            preferred_element_type=jnp.float32), 0.0)
    tab_ref[:, :64] = (
        jax.lax.dot_general(rm2, u1_ref[...], (((1,), (0,)), ((), ())),
                            preferred_element_type=jnp.float32)
        + jax.lax.dot_general(diag2, u5_ref[...], (((1,), (0,)), ((), ())),
                              preferred_element_type=jnp.float32))
    tab_ref[:, 64:] = jax.lax.dot_general(cm2, u2_ref[...], (((1,), (0,)), ((), ())),
                                          preferred_element_type=jnp.float32)

    @pl.when(i == 0)
    def _():
        gsum_ref[...] = jnp.zeros_like(gsum_ref)

    gsum_ref[...] += jnp.sum(rs, axis=0, keepdims=True)


def _tabs(rowsum2, colsum2, rc, cc, fi8rm, v1, u1, u2, u5):
    return pl.pallas_call(
        _tabs_body,
        grid=(_GN,),
        in_specs=[pl.BlockSpec((_BN, 64), lambda i: (i, 0)),
                  pl.BlockSpec((_BN, 64), lambda i: (i, 0)),
                  pl.BlockSpec((_BN, 1), lambda i: (i, 0)),
                  pl.BlockSpec((_BN, 1), lambda i: (i, 0)),
                  pl.BlockSpec((_BN, 8), lambda i: (i, 0)),
                  pl.BlockSpec((8, 64), lambda i: (0, 0)),
                  pl.BlockSpec((64, 64), lambda i: (0, 0)),
                  pl.BlockSpec((64, 64), lambda i: (0, 0)),
                  pl.BlockSpec((64, 64), lambda i: (0, 0))],
        out_specs=[pl.BlockSpec((_BN, 128), lambda i: (i, 0)),
                   pl.BlockSpec((1, 64), lambda i: (0, 0))],
        out_shape=[jax.ShapeDtypeStruct((N, 128), jnp.float32),
                   jax.ShapeDtypeStruct((1, 64), jnp.float32)],
    )(rowsum2, colsum2, rc, cc, fi8rm, v1, u1, u2, u5)


def _out2_body(fl_ref, fh_ref, tl_ref, th_ref, ec_ref, v1_ref, u0_ref, u4_ref,
               k2_ref, o_ref, s_ref):
    i = pl.program_id(0)
    v1 = v1_ref[...]
    u0 = u0_ref[...]
    u4 = u4_ref[...]
    k2 = k2_ref[...]

    def half(f_ref, t_ref, sl):
        v1a = jnp.maximum(
            jax.lax.dot_general(f_ref[...], v1, (((0,), (0,)), ((), ())),
                                preferred_element_type=jnp.float32), 0.0)
        t2 = jnp.maximum(
            jax.lax.dot_general(t_ref[...], v1, (((0,), (0,)), ((), ())),
                                preferred_element_type=jnp.float32), 0.0)
        return (jax.lax.dot_general(v1a, u0, (((1,), (0,)), ((), ())),
                                    preferred_element_type=jnp.float32)
                + jax.lax.dot_general(t2, u4, (((1,), (0,)), ((), ())),
                                      preferred_element_type=jnp.float32)
                + ec_ref[:, sl] + k2)

    olo = half(fl_ref, tl_ref, slice(0, 64))
    ohi = half(fh_ref, th_ref, slice(64, 128))
    o_ref[:, :64] = olo
    o_ref[:, 64:] = ohi

    @pl.when(i == 0)
    def _():
        s_ref[...] = jnp.zeros_like(s_ref)

    s_ref[0:1, :64] += jnp.sum(olo, axis=0, keepdims=True)
    s_ref[0:1, 64:] += jnp.sum(ohi, axis=0, keepdims=True)
    s_ref[1:2, :64] += jnp.sum(olo * olo, axis=0, keepdims=True)
    s_ref[1:2, 64:] += jnp.sum(ohi * ohi, axis=0, keepdims=True)


def _out2(f8cm, ft8cm, ecp, v1, u0, u4, k2):
    return pl.pallas_call(
        _out2_body,
        grid=(_G2,),
        in_specs=[pl.BlockSpec((8, _B2), lambda i: (0, i)),
                  pl.BlockSpec((8, _B2), lambda i: (0, i + _G2)),
                  pl.BlockSpec((8, _B2), lambda i: (0, i)),
                  pl.BlockSpec((8, _B2), lambda i: (0, i + _G2)),
                  pl.BlockSpec((_B2, 128), lambda i: (i, 0)),
                  pl.BlockSpec((8, 64), lambda i: (0, 0)),
                  pl.BlockSpec((64, 64), lambda i: (0, 0)),
                  pl.BlockSpec((64, 64), lambda i: (0, 0)),
                  pl.BlockSpec((1, 64), lambda i: (0, 0))],
        out_specs=[pl.BlockSpec((_B2, 128), lambda i: (i, 0)),
                   pl.BlockSpec((2, 128), lambda i: (0, 0))],
        out_shape=[jax.ShapeDtypeStruct((H, 128), jnp.float32),
                   jax.ShapeDtypeStruct((2, 128), jnp.float32)],
    )(f8cm, f8cm, ft8cm, ft8cm, ecp, v1, u0, u4, k2)


def _vals2_body(o_ref, m_ref, v_ref):
    o = o_ref[...]
    v_ref[...] = jnp.maximum((o - m_ref[0:1, :]) * m_ref[1:2, :], 0.0)


def _vals2(out2p, musig128):
    return pl.pallas_call(
        _vals2_body,
        grid=(_G2,),
        in_specs=[pl.BlockSpec((_B2, 128), lambda i: (i, 0)),
                  pl.BlockSpec((2, 128), lambda i: (0, 0))],
        out_specs=pl.BlockSpec((_B2, 128), lambda i: (i, 0)),
        out_shape=jax.ShapeDtypeStruct((H, 128), jnp.float32),
    )(out2p, musig128)


def _emb_body(p_ref, rc_ref, w_ref, b_ref, o_ref):
    ent = p_ref[...] / jnp.maximum(rc_ref[...], 1.0)
    o_ref[...] = (jax.lax.dot_general(ent, w_ref[...], (((1,), (0,)), ((), ())),
                                      preferred_element_type=jnp.float32)
                  + b_ref[...])


def _emb(pooled, rc, w_pool, b_pool):
    return pl.pallas_call(
        _emb_body,
        grid=(_GN,),
        in_specs=[pl.BlockSpec((_BN, 64), lambda i: (i, 0)),
                  pl.BlockSpec((_BN, 1), lambda i: (i, 0)),
                  pl.BlockSpec((64, 50), lambda i: (0, 0)),
                  pl.BlockSpec((1, 50), lambda i: (0, 0))],
        out_specs=pl.BlockSpec((_BN, 50), lambda i: (i, 0)),
        out_shape=jax.ShapeDtypeStruct((N, 50), jnp.float32),
    )(pooled, rc, w_pool, b_pool[None, :])


_EC_C = 64
_EC_NCHUNK = H // _EC_C


def _ec_body(tab_hbm, row_hbm, col_hbm, out_hbm,
             irl, irh, icl, ich, brl, brh, bcl, bch, ob, s1, s2, s3, s4):
    nc = lax.axis_size("c")
    nw = nc * lax.axis_size("s")
    wid = lax.axis_index("s") * nc + lax.axis_index("c")

    def step(it, _):
        chunk = it * nw + wid

        @pl.when(chunk < _EC_NCHUNK)
        def _():
            base = chunk * _EC_C
            pltpu.sync_copy(row_hbm.at[pl.ds(base, _EC_C)], irl)
            pltpu.sync_copy(col_hbm.at[pl.ds(base, _EC_C)], icl)
            pltpu.sync_copy(row_hbm.at[pl.ds(H + base, _EC_C)], irh)
            pltpu.sync_copy(col_hbm.at[pl.ds(H + base, _EC_C)], ich)
            cp1 = pltpu.async_copy(tab_hbm.at[irl], brl, s1)
            cp2 = pltpu.async_copy(tab_hbm.at[icl], bcl, s2)
            cp3 = pltpu.async_copy(tab_hbm.at[irh], brh, s3)
            cp4 = pltpu.async_copy(tab_hbm.at[ich], bch, s4)
            cp1.wait()
            cp2.wait()

            def addlo(j, _):
                for s in range(4):
                    ob[j, pl.ds(s * 16, 16)] = (
                        brl[j, pl.ds(s * 16, 16)] + bcl[j, pl.ds(64 + s * 16, 16)])
                return 0

            lax.fori_loop(0, _EC_C, addlo, 0, unroll=2)
            cp3.wait()
            cp4.wait()

            def addhi(j, _):
                for s in range(4):
                    ob[j, pl.ds(64 + s * 16, 16)] = (
                        brh[j, pl.ds(s * 16, 16)] + bch[j, pl.ds(64 + s * 16, 16)])
                return 0

            lax.fori_loop(0, _EC_C, addhi, 0, unroll=2)
            pltpu.sync_copy(ob, out_hbm.at[pl.ds(base, _EC_C)])
        return 0

    lax.fori_loop(0, (_EC_NCHUNK + 31) // 32, step, 0)


def _ec_gather(tab, row, col):
    return pl.kernel(
        _ec_body,
        out_type=jax.ShapeDtypeStruct((H, 128), jnp.float32),
        mesh=plsc.VectorSubcoreMesh(core_axis_name="c", subcore_axis_name="s"),
        scratch_types=[
            pltpu.VMEM((_EC_C,), jnp.int32),
            pltpu.VMEM((_EC_C,), jnp.int32),
            pltpu.VMEM((_EC_C,), jnp.int32),
            pltpu.VMEM((_EC_C,), jnp.int32),
            pltpu.VMEM((_EC_C, 128), jnp.float32),
            pltpu.VMEM((_EC_C, 128), jnp.float32),
            pltpu.VMEM((_EC_C, 128), jnp.float32),
            pltpu.VMEM((_EC_C, 128), jnp.float32),
            pltpu.VMEM((_EC_C, 128), jnp.float32),
            pltpu.SemaphoreType.DMA,
            pltpu.SemaphoreType.DMA,
            pltpu.SemaphoreType.DMA,
            pltpu.SemaphoreType.DMA,
        ],
    )(tab, row, col)


_SS_C = 128
_SS_H = 25016


def _segsum_body(vals_hbm, idx_hbm, out_hbm, vbuf, idxv, idxh, tbl, zb):
    cid = lax.axis_index("c")
    sid = lax.axis_index("s")
    nchunk = NNZ // _SS_C

    def zrow(j, _):
        for s in range(2):
            zb[j, pl.ds(s * 16, 16)] = jnp.zeros((16,), jnp.float32)
        return 0

    lax.fori_loop(0, 256, zrow, 0)

    nz = (2 * _SS_H) // 256

    def zcp(j, _):
        @pl.when(j * 16 + sid < nz)
        def _():
            pltpu.sync_copy(zb, tbl.at[pl.ds((j * 16 + sid) * 256, 256)])
        return 0

    lax.fori_loop(0, (nz + 15) // 16, zcp, 0)

    @pl.when(sid == 0)
    def _():
        pltpu.sync_copy(zb.at[pl.ds(0, 2 * _SS_H - nz * 256)],
                        tbl.at[pl.ds(nz * 256, 2 * _SS_H - nz * 256)])

    plsc.subcore_barrier()

    lane = lax.iota(jnp.int32, 16)

    def step(it, _):
        chunk = it * 16 + sid

        @pl.when(chunk < nchunk)
        def _():
            base = chunk * _SS_C
            pltpu.sync_copy(idx_hbm.at[pl.ds(base, _SS_C)], idxv)
            pltpu.sync_copy(vals_hbm.at[pl.ds(base * 2, 2 * _SS_C)], vbuf)

            def mkidx(j, _):
                v = idxv[pl.ds(j * 16, 16)]
                loc = v - cid * 25000
                oob = (loc < 0) | (loc >= 25000)
                loc = jnp.where(oob, 25000 + ((lane + j) & 15), loc)
                plsc.store_scatter(idxh, [2 * lane + 32 * j], 2 * loc)
                plsc.store_scatter(idxh, [2 * lane + 32 * j + 1], 2 * loc + 1)
                return 0

            lax.fori_loop(0, _SS_C // 16, mkidx, 0)
            pltpu.sync_copy(vbuf, tbl.at[idxh], add=True)
        return 0

    lax.fori_loop(0, (nchunk + 15) // 16, step, 0)
    plsc.subcore_barrier()
    pltpu.sync_copy(tbl.at[pl.ds(sid * 3125, 3125)],
                    out_hbm.at[pl.ds(cid * 50000 + sid * 3125, 3125)])


def _segsum(vals_p, idx_perm):
    v2 = jnp.reshape(vals_p, (2 * NNZ, 32))
    out = pl.kernel(
        _segsum_body,
        out_type=jax.ShapeDtypeStruct((2 * N, 32), jnp.float32),
        mesh=plsc.VectorSubcoreMesh(core_axis_name="c", subcore_axis_name="s"),
        scratch_types=[
            pltpu.VMEM((2 * _SS_C, 32), jnp.float32),
            pltpu.VMEM((_SS_C,), jnp.int32),
            pltpu.VMEM((2 * _SS_C,), jnp.int32),
            pltpu.VMEM_SHARED((2 * _SS_H, 32), jnp.float32),
            pltpu.VMEM((256, 32), jnp.float32),
        ],
        compiler_params=pltpu.CompilerParams(use_tc_tiling_on_sc=False,
                                             needs_layout_passes=False),
    )(v2, idx_perm)
    return jnp.reshape(out, (N, 64))


def _pairperm(x):
    return jnp.stack([x[:H], x[H:]], axis=1).reshape(-1)


def kernel(data_values, data_indices, idx_identity, idx_transpose, W_in, b_in,
           Ws, bs, W_pool, b_pool):
    d = data_values[:, 0]
    row = data_indices[0]
    col = data_indices[1]
    ones = jnp.ones((NNZ,), jnp.float32)

    rs = jax.ops.segment_sum(d, row, num_segments=N)
    rc = jax.ops.segment_sum(ones, row, num_segments=N)
    cs = jax.ops.segment_sum(d, col, num_segments=N)
    cc = jax.ops.segment_sum(ones, col, num_segments=N)
    rmd = rs / jnp.maximum(rc, 1.0)
    mr = jnp.minimum(rc, 1.0)
    cmd = cs / jnp.maximum(cc, 1.0)
    mc = jnp.minimum(cc, 1.0)
    dt = d[idx_transpose]
    di = d[idx_identity]

    w = W_in[0]
    b = b_in
    W0, W1, W2, W3, W4, W5 = [Ws[0, i] for i in range(6)]
    gm = jnp.sum(rs) / NNZ
    k = gm * (w @ W3) + b @ (W0 + W3 + W4 + W5) + bs[0]
    V8 = jnp.stack([w @ W0, w @ W4, w @ W1, w @ W2, w @ W5, b @ W1, b @ W2, k],
                   axis=0)

    f8cm = jnp.stack([d, dt, rmd[row], cmd[col], di[row], mr[row], mc[col],
                      ones], axis=0)

    G = _gram(f8cm)
    mA = G[7, :] / NNZ
    mu1 = mA @ V8
    Eo2 = jnp.einsum('ic,ij,jc->c', V8, G / NNZ, V8)
    var1 = Eo2 - mu1 * mu1
    sig1 = jnp.sqrt(var1 + EPS)
    V1 = (V8 - jnp.eye(8, dtype=jnp.float32)[:, 7:8] * mu1[None, :]) / sig1[None, :]

    vals1p = _vals1(f8cm, V1)
    rowp = _pairperm(row)
    colp = _pairperm(col)
    rowsum2 = _segsum(vals1p, rowp)
    colsum2 = _segsum(vals1p, colp)

    ft8cm = jnp.take(f8cm, idx_transpose, axis=1)
    fi8rm = jnp.take(f8cm, idx_identity, axis=1).T

    U0, U1, U2, U3, U4, U5 = [Ws[1, i] for i in range(6)]
    tab, gsum = _tabs(rowsum2, colsum2, rc[:, None], cc[:, None], fi8rm,
                      V1, U1, U2, U5)
    g2 = gsum[0] / NNZ
    k2 = (g2 @ U3 + bs[1])[None, :]

    ecp = _ec_gather(tab, row, col)

    out2p, s12 = _out2(f8cm, ft8cm, ecp, V1, U0, U4, k2)
    mu2 = (s12[0, :64] + s12[0, 64:]) / NNZ
    e2 = (s12[1, :64] + s12[1, 64:]) / NNZ
    var2 = e2 - mu2 * mu2
    ms = jnp.stack([mu2, 1.0 / jnp.sqrt(var2 + EPS)], axis=0)
    musig128 = jnp.concatenate([ms, ms], axis=1)
    vals2p = _vals2(out2p, musig128)
    pooled = _segsum(vals2p, rowp)
    return _emb(pooled, rc[:, None], W_pool, b_pool)


# speedup vs baseline: 1.4366x; 1.2303x over previous
import functools
import jax
import jax.numpy as jnp
from jax import lax
from jax.experimental import pallas as pl
from jax.experimental.pallas import tpu as pltpu
from jax.experimental.pallas import tpu_sc as plsc

N = 50000
NNZ = 800000
H = NNZ // 2
EPS = 1e-5

_BE = 6400
_B2 = 3200
_BN = 2000
_GE = NNZ // _BE
_G2 = H // _B2
_GN = N // _BN


def _gram_body(f_ref, g_ref):
    i = pl.program_id(0)

    @pl.when(i == 0)
    def _():
        g_ref[...] = jnp.zeros_like(g_ref)

    f = f_ref[...]
    g_ref[...] += jax.lax.dot_general(f, f, (((1,), (1,)), ((), ())),
                                      preferred_element_type=jnp.float32)


def _gram(f8cm):
    return pl.pallas_call(
        _gram_body,
        grid=(_GE,),
        in_specs=[pl.BlockSpec((8, _BE), lambda i: (0, i))],
        out_specs=pl.BlockSpec((8, 8), lambda i: (0, 0)),
        out_shape=jax.ShapeDtypeStruct((8, 8), jnp.float32),
    )(f8cm)


def _vals1_body(fl_ref, fh_ref, v_ref, o_ref):
    v = v_ref[...]
    o_ref[:, :64] = jnp.maximum(
        jax.lax.dot_general(fl_ref[...], v, (((0,), (0,)), ((), ())),
                            preferred_element_type=jnp.float32), 0.0)
    o_ref[:, 64:] = jnp.maximum(
        jax.lax.dot_general(fh_ref[...], v, (((0,), (0,)), ((), ())),
                            preferred_element_type=jnp.float32), 0.0)


def _vals1(f8cm, v1):
    return pl.pallas_call(
        _vals1_body,
        grid=(_G2,),
        in_specs=[pl.BlockSpec((8, _B2), lambda i: (0, i)),
                  pl.BlockSpec((8, _B2), lambda i: (0, i + _G2)),
                  pl.BlockSpec((8, 64), lambda i: (0, 0))],
        out_specs=pl.BlockSpec((_B2, 128), lambda i: (i, 0)),
        out_shape=jax.ShapeDtypeStruct((H, 128), jnp.float32),
    )(f8cm, f8cm, v1)


def _tabs_body(rs_ref, cs_ref, rc_ref, cc_ref, fi_ref, v1_ref, u1_ref, u2_ref,
               u5_ref, tab_ref, gsum_ref):
    i = pl.program_id(0)
    rc = jnp.maximum(rc_ref[...], 1.0)
    cc = jnp.maximum(cc_ref[...], 1.0)
    rs = rs_ref[...]
    rm2 = rs / rc
    cm2 = cs_ref[...] / cc
    diag2 = jnp.maximum(
        jax.lax.dot_general(fi_ref[...], v1_ref[...], (((1,), (0,)), ((), ())),
                            preferred_element_type=jnp.float32), 0.0)
    tab_ref[:, :64] = (
        jax.lax.dot_general(rm2, u1_ref[...], (((1,), (0,)), ((), ())),
                            preferred_element_type=jnp.float32)
        + jax.lax.dot_general(diag2, u5_ref[...], (((1,), (0,)), ((), ())),
                              preferred_element_type=jnp.float32))
    tab_ref[:, 64:] = jax.lax.dot_general(cm2, u2_ref[...], (((1,), (0,)), ((), ())),
                                          preferred_element_type=jnp.float32)

    @pl.when(i == 0)
    def _():
        gsum_ref[...] = jnp.zeros_like(gsum_ref)

    gsum_ref[...] += jnp.sum(rs, axis=0, keepdims=True)


def _tabs(rowsum2, colsum2, rc, cc, fi8rm, v1, u1, u2, u5):
    return pl.pallas_call(
        _tabs_body,
        grid=(_GN,),
        in_specs=[pl.BlockSpec((_BN, 64), lambda i: (i, 0)),
                  pl.BlockSpec((_BN, 64), lambda i: (i, 0)),
                  pl.BlockSpec((_BN, 1), lambda i: (i, 0)),
                  pl.BlockSpec((_BN, 1), lambda i: (i, 0)),
                  pl.BlockSpec((_BN, 8), lambda i: (i, 0)),
                  pl.BlockSpec((8, 64), lambda i: (0, 0)),
                  pl.BlockSpec((64, 64), lambda i: (0, 0)),
                  pl.BlockSpec((64, 64), lambda i: (0, 0)),
                  pl.BlockSpec((64, 64), lambda i: (0, 0))],
        out_specs=[pl.BlockSpec((_BN, 128), lambda i: (i, 0)),
                   pl.BlockSpec((1, 64), lambda i: (0, 0))],
        out_shape=[jax.ShapeDtypeStruct((N, 128), jnp.float32),
                   jax.ShapeDtypeStruct((1, 64), jnp.float32)],
    )(rowsum2, colsum2, rc, cc, fi8rm, v1, u1, u2, u5)


def _out2_body(fl_ref, fh_ref, tl_ref, th_ref, ec_ref, v1_ref, u0_ref, u4_ref,
               k2_ref, o_ref, s_ref):
    i = pl.program_id(0)
    v1 = v1_ref[...]
    u0 = u0_ref[...]
    u4 = u4_ref[...]
    k2 = k2_ref[...]

    def half(f_ref, t_ref, sl):
        v1a = jnp.maximum(
            jax.lax.dot_general(f_ref[...], v1, (((0,), (0,)), ((), ())),
                                preferred_element_type=jnp.float32), 0.0)
        t2 = jnp.maximum(
            jax.lax.dot_general(t_ref[...], v1, (((0,), (0,)), ((), ())),
                                preferred_element_type=jnp.float32), 0.0)
        return (jax.lax.dot_general(v1a, u0, (((1,), (0,)), ((), ())),
                                    preferred_element_type=jnp.float32)
                + jax.lax.dot_general(t2, u4, (((1,), (0,)), ((), ())),
                                      preferred_element_type=jnp.float32)
                + ec_ref[:, sl] + k2)

    olo = half(fl_ref, tl_ref, slice(0, 64))
    ohi = half(fh_ref, th_ref, slice(64, 128))
    o_ref[:, :64] = olo
    o_ref[:, 64:] = ohi

    @pl.when(i == 0)
    def _():
        s_ref[...] = jnp.zeros_like(s_ref)

    s_ref[0:1, :64] += jnp.sum(olo, axis=0, keepdims=True)
    s_ref[0:1, 64:] += jnp.sum(ohi, axis=0, keepdims=True)
    s_ref[1:2, :64] += jnp.sum(olo * olo, axis=0, keepdims=True)
    s_ref[1:2, 64:] += jnp.sum(ohi * ohi, axis=0, keepdims=True)


def _out2(f8cm, ft8cm, ecp, v1, u0, u4, k2):
    return pl.pallas_call(
        _out2_body,
        grid=(_G2,),
        in_specs=[pl.BlockSpec((8, _B2), lambda i: (0, i)),
                  pl.BlockSpec((8, _B2), lambda i: (0, i + _G2)),
                  pl.BlockSpec((8, _B2), lambda i: (0, i)),
                  pl.BlockSpec((8, _B2), lambda i: (0, i + _G2)),
                  pl.BlockSpec((_B2, 128), lambda i: (i, 0)),
                  pl.BlockSpec((8, 64), lambda i: (0, 0)),
                  pl.BlockSpec((64, 64), lambda i: (0, 0)),
                  pl.BlockSpec((64, 64), lambda i: (0, 0)),
                  pl.BlockSpec((1, 64), lambda i: (0, 0))],
        out_specs=[pl.BlockSpec((_B2, 128), lambda i: (i, 0)),
                   pl.BlockSpec((2, 128), lambda i: (0, 0))],
        out_shape=[jax.ShapeDtypeStruct((H, 128), jnp.float32),
                   jax.ShapeDtypeStruct((2, 128), jnp.float32)],
    )(f8cm, f8cm, ft8cm, ft8cm, ecp, v1, u0, u4, k2)


def _vals2_body(o_ref, m_ref, v_ref):
    o = o_ref[...]
    v_ref[...] = jnp.maximum((o - m_ref[0:1, :]) * m_ref[1:2, :], 0.0)


def _vals2(out2p, musig128):
    return pl.pallas_call(
        _vals2_body,
        grid=(_G2,),
        in_specs=[pl.BlockSpec((_B2, 128), lambda i: (i, 0)),
                  pl.BlockSpec((2, 128), lambda i: (0, 0))],
        out_specs=pl.BlockSpec((_B2, 128), lambda i: (i, 0)),
        out_shape=jax.ShapeDtypeStruct((H, 128), jnp.float32),
    )(out2p, musig128)


def _emb_body(p_ref, rc_ref, w_ref, b_ref, o_ref):
    ent = p_ref[...] / jnp.maximum(rc_ref[...], 1.0)
    o_ref[...] = (jax.lax.dot_general(ent, w_ref[...], (((1,), (0,)), ((), ())),
                                      preferred_element_type=jnp.float32)
                  + b_ref[...])


def _emb(pooled, rc, w_pool, b_pool):
    return pl.pallas_call(
        _emb_body,
        grid=(_GN,),
        in_specs=[pl.BlockSpec((_BN, 64), lambda i: (i, 0)),
                  pl.BlockSpec((_BN, 1), lambda i: (i, 0)),
                  pl.BlockSpec((64, 50), lambda i: (0, 0)),
                  pl.BlockSpec((1, 50), lambda i: (0, 0))],
        out_specs=pl.BlockSpec((_BN, 50), lambda i: (i, 0)),
        out_shape=jax.ShapeDtypeStruct((N, 50), jnp.float32),
    )(pooled, rc, w_pool, b_pool[None, :])


_EC_C = 64
_EC_NCHUNK = H // _EC_C


def _ec_body(tab_hbm, row_hbm, col_hbm, out_hbm,
             irl, irh, icl, ich, brl, brh, bcl, bch, ob, s1, s2, s3, s4):
    nc = lax.axis_size("c")
    nw = nc * lax.axis_size("s")
    wid = lax.axis_index("s") * nc + lax.axis_index("c")

    def step(it, _):
        chunk = it * nw + wid

        @pl.when(chunk < _EC_NCHUNK)
        def _():
            base = chunk * _EC_C
            pltpu.sync_copy(row_hbm.at[pl.ds(base, _EC_C)], irl)
            pltpu.sync_copy(col_hbm.at[pl.ds(base, _EC_C)], icl)
            pltpu.sync_copy(row_hbm.at[pl.ds(H + base, _EC_C)], irh)
            pltpu.sync_copy(col_hbm.at[pl.ds(H + base, _EC_C)], ich)
            cp1 = pltpu.async_copy(tab_hbm.at[irl], brl, s1)
            cp2 = pltpu.async_copy(tab_hbm.at[icl], bcl, s2)
            cp3 = pltpu.async_copy(tab_hbm.at[irh], brh, s3)
            cp4 = pltpu.async_copy(tab_hbm.at[ich], bch, s4)
            cp1.wait()
            cp2.wait()

            def addlo(j, _):
                for s in range(4):
                    ob[j, pl.ds(s * 16, 16)] = (
                        brl[j, pl.ds(s * 16, 16)] + bcl[j, pl.ds(64 + s * 16, 16)])
                return 0

            lax.fori_loop(0, _EC_C, addlo, 0, unroll=2)
            cp3.wait()
            cp4.wait()

            def addhi(j, _):
                for s in range(4):
                    ob[j, pl.ds(64 + s * 16, 16)] = (
                        brh[j, pl.ds(s * 16, 16)] + bch[j, pl.ds(64 + s * 16, 16)])
                return 0

            lax.fori_loop(0, _EC_C, addhi, 0, unroll=2)
            pltpu.sync_copy(ob, out_hbm.at[pl.ds(base, _EC_C)])
        return 0

    lax.fori_loop(0, (_EC_NCHUNK + 31) // 32, step, 0)


def _ec_gather(tab, row, col):
    return pl.kernel(
        _ec_body,
        out_type=jax.ShapeDtypeStruct((H, 128), jnp.float32),
        mesh=plsc.VectorSubcoreMesh(core_axis_name="c", subcore_axis_name="s"),
        scratch_types=[
            pltpu.VMEM((_EC_C,), jnp.int32),
            pltpu.VMEM((_EC_C,), jnp.int32),
            pltpu.VMEM((_EC_C,), jnp.int32),
            pltpu.VMEM((_EC_C,), jnp.int32),
            pltpu.VMEM((_EC_C, 128), jnp.float32),
            pltpu.VMEM((_EC_C, 128), jnp.float32),
            pltpu.VMEM((_EC_C, 128), jnp.float32),
            pltpu.VMEM((_EC_C, 128), jnp.float32),
            pltpu.VMEM((_EC_C, 128), jnp.float32),
            pltpu.SemaphoreType.DMA,
            pltpu.SemaphoreType.DMA,
            pltpu.SemaphoreType.DMA,
            pltpu.SemaphoreType.DMA,
        ],
    )(tab, row, col)


_SS_C = 128
_SS_H = 25016


def _segsum_body(vals_hbm, idx_hbm, out_hbm, vbuf, idxv, idxh, tbl, zb):
    cid = lax.axis_index("c")
    sid = lax.axis_index("s")
    nchunk = NNZ // _SS_C

    def zrow(j, _):
        for s in range(2):
            zb[j, pl.ds(s * 16, 16)] = jnp.zeros((16,), jnp.float32)
        return 0

    lax.fori_loop(0, 256, zrow, 0)

    nz = (2 * _SS_H) // 256

    def zcp(j, _):
        @pl.when(j * 16 + sid < nz)
        def _():
            pltpu.sync_copy(zb, tbl.at[pl.ds((j * 16 + sid) * 256, 256)])
        return 0

    lax.fori_loop(0, (nz + 15) // 16, zcp, 0)

    @pl.when(sid == 0)
    def _():
        pltpu.sync_copy(zb.at[pl.ds(0, 2 * _SS_H - nz * 256)],
                        tbl.at[pl.ds(nz * 256, 2 * _SS_H - nz * 256)])

    plsc.subcore_barrier()

    lane = lax.iota(jnp.int32, 16)

    def step(it, _):
        chunk = it * 16 + sid

        @pl.when(chunk < nchunk)
        def _():
            base = chunk * _SS_C
            pltpu.sync_copy(idx_hbm.at[pl.ds(base, _SS_C)], idxv)
            pltpu.sync_copy(vals_hbm.at[pl.ds(base * 2, 2 * _SS_C)], vbuf)

            def mkidx(j, _):
                v = idxv[pl.ds(j * 16, 16)]
                loc = v - cid * 25000
                oob = (loc < 0) | (loc >= 25000)
                loc = jnp.where(oob, 25000 + ((lane + j) & 15), loc)
                plsc.store_scatter(idxh, [2 * lane + 32 * j], 2 * loc)
                plsc.store_scatter(idxh, [2 * lane + 32 * j + 1], 2 * loc + 1)
                return 0

            lax.fori_loop(0, _SS_C // 16, mkidx, 0)
            pltpu.sync_copy(vbuf, tbl.at[idxh], add=True)
        return 0

    lax.fori_loop(0, (nchunk + 15) // 16, step, 0)
    plsc.subcore_barrier()
    pltpu.sync_copy(tbl.at[pl.ds(sid * 3125, 3125)],
                    out_hbm.at[pl.ds(cid * 50000 + sid * 3125, 3125)])


def _segsum(vals_p, idx_perm):
    v2 = jnp.reshape(vals_p, (2 * NNZ, 32))
    out = pl.kernel(
        _segsum_body,
        out_type=jax.ShapeDtypeStruct((2 * N, 32), jnp.float32),
        mesh=plsc.VectorSubcoreMesh(core_axis_name="c", subcore_axis_name="s"),
        scratch_types=[
            pltpu.VMEM((2 * _SS_C, 32), jnp.float32),
            pltpu.VMEM((_SS_C,), jnp.int32),
            pltpu.VMEM((2 * _SS_C,), jnp.int32),
            pltpu.VMEM_SHARED((2 * _SS_H, 32), jnp.float32),
            pltpu.VMEM((256, 32), jnp.float32),
        ],
        compiler_params=pltpu.CompilerParams(use_tc_tiling_on_sc=False,
                                             needs_layout_passes=False),
    )(v2, idx_perm)
    return jnp.reshape(out, (N, 64))


def _pairperm(x):
    return jnp.stack([x[:H], x[H:]], axis=1).reshape(-1)


def kernel(data_values, data_indices, idx_identity, idx_transpose, W_in, b_in,
           Ws, bs, W_pool, b_pool):
    d = data_values[:, 0]
    row = data_indices[0]
    col = data_indices[1]
    ones = jnp.ones((NNZ,), jnp.float32)

    rs = jax.ops.segment_sum(d, row, num_segments=N)
    rc = jax.ops.segment_sum(ones, row, num_segments=N)
    cs = jax.ops.segment_sum(d, col, num_segments=N)
    cc = jax.ops.segment_sum(ones, col, num_segments=N)
    rmd = rs / jnp.maximum(rc, 1.0)
    mr = jnp.minimum(rc, 1.0)
    cmd = cs / jnp.maximum(cc, 1.0)
    mc = jnp.minimum(cc, 1.0)
    dt = d[idx_transpose]
    di = d[idx_identity]

    w = W_in[0]
    b = b_in
    W0, W1, W2, W3, W4, W5 = [Ws[0, i] for i in range(6)]
    gm = jnp.sum(rs) / NNZ
    k = gm * (w @ W3) + b @ (W0 + W3 + W4 + W5) + bs[0]
    V8 = jnp.stack([w @ W0, w @ W4, w @ W1, w @ W2, w @ W5, b @ W1, b @ W2, k],
                   axis=0)

    feats = [d, dt, rmd[row], cmd[col], di[row], mr[row], mc[col], ones]
    f8cm = jnp.stack(feats, axis=0)

    G = _gram(f8cm)
    mA = G[7, :] / NNZ
    mu1 = mA @ V8
    Eo2 = jnp.einsum('ic,ij,jc->c', V8, G / NNZ, V8)
    var1 = Eo2 - mu1 * mu1
    sig1 = jnp.sqrt(var1 + EPS)
    V1 = (V8 - jnp.eye(8, dtype=jnp.float32)[:, 7:8] * mu1[None, :]) / sig1[None, :]

    vals1p = _vals1(f8cm, V1)
    rowp = _pairperm(row)
    colp = _pairperm(col)
    rowsum2 = _segsum(vals1p, rowp)
    colsum2 = _segsum(vals1p, colp)

    ft8cm = jnp.stack([dt] + [jnp.take(f, idx_transpose) for f in feats[1:7]]
                      + [ones], axis=0)
    fi8rm = jnp.stack([jnp.take(f, idx_identity) for f in feats[:7]]
                      + [jnp.ones((N,), jnp.float32)], axis=1)

    U0, U1, U2, U3, U4, U5 = [Ws[1, i] for i in range(6)]
    tab, gsum = _tabs(rowsum2, colsum2, rc[:, None], cc[:, None], fi8rm,
                      V1, U1, U2, U5)
    g2 = gsum[0] / NNZ
    k2 = (g2 @ U3 + bs[1])[None, :]

    ecp = _ec_gather(tab, row, col)

    out2p, s12 = _out2(f8cm, ft8cm, ecp, V1, U0, U4, k2)
    mu2 = (s12[0, :64] + s12[0, 64:]) / NNZ
    e2 = (s12[1, :64] + s12[1, 64:]) / NNZ
    var2 = e2 - mu2 * mu2
    ms = jnp.stack([mu2, 1.0 / jnp.sqrt(var2 + EPS)], axis=0)
    musig128 = jnp.concatenate([ms, ms], axis=1)
    vals2p = _vals2(out2p, musig128)
    pooled = _segsum(vals2p, rowp)
    return _emb(pooled, rc[:, None], W_pool, b_pool)


# speedup vs baseline: 2.1726x; 1.5123x over previous
import functools
import jax
import jax.numpy as jnp
from jax import lax
from jax.experimental import pallas as pl
from jax.experimental.pallas import tpu as pltpu
from jax.experimental.pallas import tpu_sc as plsc

N = 50000
NNZ = 800000
H = NNZ // 2
EPS = 1e-5

_BE = 6400
_B2 = 3200
_BN = 2000
_GE = NNZ // _BE
_G2 = H // _B2
_GN = N // _BN


def _gram_body(f_ref, g_ref):
    i = pl.program_id(0)

    @pl.when(i == 0)
    def _():
        g_ref[...] = jnp.zeros_like(g_ref)

    f = f_ref[...]
    g_ref[...] += jax.lax.dot_general(f, f, (((1,), (1,)), ((), ())),
                                      preferred_element_type=jnp.float32)


def _gram(f8cm):
    return pl.pallas_call(
        _gram_body,
        grid=(_GE,),
        in_specs=[pl.BlockSpec((8, _BE), lambda i: (0, i))],
        out_specs=pl.BlockSpec((8, 8), lambda i: (0, 0)),
        out_shape=jax.ShapeDtypeStruct((8, 8), jnp.float32),
    )(f8cm)


def _vals1_body(fl_ref, fh_ref, v_ref, o_ref):
    v = v_ref[...]
    o_ref[:, :64] = jnp.maximum(
        jax.lax.dot_general(fl_ref[...], v, (((0,), (0,)), ((), ())),
                            preferred_element_type=jnp.float32), 0.0)
    o_ref[:, 64:] = jnp.maximum(
        jax.lax.dot_general(fh_ref[...], v, (((0,), (0,)), ((), ())),
                            preferred_element_type=jnp.float32), 0.0)


def _vals1(f8cm, v1):
    return pl.pallas_call(
        _vals1_body,
        grid=(_G2,),
        in_specs=[pl.BlockSpec((8, _B2), lambda i: (0, i)),
                  pl.BlockSpec((8, _B2), lambda i: (0, i + _G2)),
                  pl.BlockSpec((8, 64), lambda i: (0, 0))],
        out_specs=pl.BlockSpec((_B2, 128), lambda i: (i, 0)),
        out_shape=jax.ShapeDtypeStruct((H, 128), jnp.float32),
    )(f8cm, f8cm, v1)


def _tabs_body(rs_ref, cs_ref, rc_ref, cc_ref, fi_ref, v1_ref, u1_ref, u2_ref,
               u5_ref, tab_ref, gsum_ref):
    i = pl.program_id(0)
    rc = jnp.maximum(rc_ref[...], 1.0)
    cc = jnp.maximum(cc_ref[...], 1.0)
    rs = rs_ref[...]
    rm2 = rs / rc
    cm2 = cs_ref[...] / cc
    diag2 = jnp.maximum(
        jax.lax.dot_general(fi_ref[...], v1_ref[...], (((1,), (0,)), ((), ())),
                            preferred_element_type=jnp.float32), 0.0)
    tab_ref[:, :64] = (
        jax.lax.dot_general(rm2, u1_ref[...], (((1,), (0,)), ((), ())),
                            preferred_element_type=jnp.float32)
        + jax.lax.dot_general(diag2, u5_ref[...], (((1,), (0,)), ((), ())),
                              preferred_element_type=jnp.float32))
    tab_ref[:, 64:] = jax.lax.dot_general(cm2, u2_ref[...], (((1,), (0,)), ((), ())),
                                          preferred_element_type=jnp.float32)

    @pl.when(i == 0)
    def _():
        gsum_ref[...] = jnp.zeros_like(gsum_ref)

    gsum_ref[...] += jnp.sum(rs, axis=0, keepdims=True)


def _tabs(rowsum2, colsum2, rc, cc, fi8rm, v1, u1, u2, u5):
    return pl.pallas_call(
        _tabs_body,
        grid=(_GN,),
        in_specs=[pl.BlockSpec((_BN, 64), lambda i: (i, 0)),
                  pl.BlockSpec((_BN, 64), lambda i: (i, 0)),
                  pl.BlockSpec((_BN, 1), lambda i: (i, 0)),
                  pl.BlockSpec((_BN, 1), lambda i: (i, 0)),
                  pl.BlockSpec((_BN, 8), lambda i: (i, 0)),
                  pl.BlockSpec((8, 64), lambda i: (0, 0)),
                  pl.BlockSpec((64, 64), lambda i: (0, 0)),
                  pl.BlockSpec((64, 64), lambda i: (0, 0)),
                  pl.BlockSpec((64, 64), lambda i: (0, 0))],
        out_specs=[pl.BlockSpec((_BN, 128), lambda i: (i, 0)),
                   pl.BlockSpec((1, 64), lambda i: (0, 0))],
        out_shape=[jax.ShapeDtypeStruct((N, 128), jnp.float32),
                   jax.ShapeDtypeStruct((1, 64), jnp.float32)],
    )(rowsum2, colsum2, rc, cc, fi8rm, v1, u1, u2, u5)


def _out2_body(fl_ref, fh_ref, tl_ref, th_ref, ec_ref, v1_ref, u0_ref, u4_ref,
               k2_ref, o_ref, s_ref):
    i = pl.program_id(0)
    v1 = v1_ref[...]
    u0 = u0_ref[...]
    u4 = u4_ref[...]
    k2 = k2_ref[...]

    def half(f_ref, t_ref, sl):
        v1a = jnp.maximum(
            jax.lax.dot_general(f_ref[...], v1, (((0,), (0,)), ((), ())),
                                preferred_element_type=jnp.float32), 0.0)
        t2 = jnp.maximum(
            jax.lax.dot_general(t_ref[...], v1, (((0,), (0,)), ((), ())),
                                preferred_element_type=jnp.float32), 0.0)
        return (jax.lax.dot_general(v1a, u0, (((1,), (0,)), ((), ())),
                                    preferred_element_type=jnp.float32)
                + jax.lax.dot_general(t2, u4, (((1,), (0,)), ((), ())),
                                      preferred_element_type=jnp.float32)
                + ec_ref[:, sl] + k2)

    olo = half(fl_ref, tl_ref, slice(0, 64))
    ohi = half(fh_ref, th_ref, slice(64, 128))
    o_ref[:, :64] = olo
    o_ref[:, 64:] = ohi

    @pl.when(i == 0)
    def _():
        s_ref[...] = jnp.zeros_like(s_ref)

    s_ref[0:1, :64] += jnp.sum(olo, axis=0, keepdims=True)
    s_ref[0:1, 64:] += jnp.sum(ohi, axis=0, keepdims=True)
    s_ref[1:2, :64] += jnp.sum(olo * olo, axis=0, keepdims=True)
    s_ref[1:2, 64:] += jnp.sum(ohi * ohi, axis=0, keepdims=True)


def _out2(f8cm, ft8cm, ecp, v1, u0, u4, k2):
    return pl.pallas_call(
        _out2_body,
        grid=(_G2,),
        in_specs=[pl.BlockSpec((8, _B2), lambda i: (0, i)),
                  pl.BlockSpec((8, _B2), lambda i: (0, i + _G2)),
                  pl.BlockSpec((8, _B2), lambda i: (0, i)),
                  pl.BlockSpec((8, _B2), lambda i: (0, i + _G2)),
                  pl.BlockSpec((_B2, 128), lambda i: (i, 0)),
                  pl.BlockSpec((8, 64), lambda i: (0, 0)),
                  pl.BlockSpec((64, 64), lambda i: (0, 0)),
                  pl.BlockSpec((64, 64), lambda i: (0, 0)),
                  pl.BlockSpec((1, 64), lambda i: (0, 0))],
        out_specs=[pl.BlockSpec((_B2, 128), lambda i: (i, 0)),
                   pl.BlockSpec((2, 128), lambda i: (0, 0))],
        out_shape=[jax.ShapeDtypeStruct((H, 128), jnp.float32),
                   jax.ShapeDtypeStruct((2, 128), jnp.float32)],
    )(f8cm, f8cm, ft8cm, ft8cm, ecp, v1, u0, u4, k2)


def _vals2_body(o_ref, m_ref, v_ref):
    o = o_ref[...]
    v_ref[...] = jnp.maximum((o - m_ref[0:1, :]) * m_ref[1:2, :], 0.0)


def _vals2(out2p, musig128):
    return pl.pallas_call(
        _vals2_body,
        grid=(_G2,),
        in_specs=[pl.BlockSpec((_B2, 128), lambda i: (i, 0)),
                  pl.BlockSpec((2, 128), lambda i: (0, 0))],
        out_specs=pl.BlockSpec((_B2, 128), lambda i: (i, 0)),
        out_shape=jax.ShapeDtypeStruct((H, 128), jnp.float32),
    )(out2p, musig128)


def _emb_body(p_ref, rc_ref, w_ref, b_ref, o_ref):
    ent = p_ref[...] / jnp.maximum(rc_ref[...], 1.0)
    o_ref[...] = (jax.lax.dot_general(ent, w_ref[...], (((1,), (0,)), ((), ())),
                                      preferred_element_type=jnp.float32)
                  + b_ref[...])


def _emb(pooled, rc, w_pool, b_pool):
    return pl.pallas_call(
        _emb_body,
        grid=(_GN,),
        in_specs=[pl.BlockSpec((_BN, 64), lambda i: (i, 0)),
                  pl.BlockSpec((_BN, 1), lambda i: (i, 0)),
                  pl.BlockSpec((64, 50), lambda i: (0, 0)),
                  pl.BlockSpec((1, 50), lambda i: (0, 0))],
        out_specs=pl.BlockSpec((_BN, 50), lambda i: (i, 0)),
        out_shape=jax.ShapeDtypeStruct((N, 50), jnp.float32),
    )(pooled, rc, w_pool, b_pool[None, :])


_EC_C = 64
_EC_NCHUNK = H // _EC_C


def _ec_body(tab_hbm, row_hbm, col_hbm, out_hbm,
             irl, irh, icl, ich, brl, brh, bcl, bch, ob, s1, s2, s3, s4):
    nc = lax.axis_size("c")
    nw = nc * lax.axis_size("s")
    wid = lax.axis_index("s") * nc + lax.axis_index("c")

    def step(it, _):
        chunk = it * nw + wid

        @pl.when(chunk < _EC_NCHUNK)
        def _():
            base = chunk * _EC_C
            pltpu.sync_copy(row_hbm.at[pl.ds(base, _EC_C)], irl)
            pltpu.sync_copy(col_hbm.at[pl.ds(base, _EC_C)], icl)
            pltpu.sync_copy(row_hbm.at[pl.ds(H + base, _EC_C)], irh)
            pltpu.sync_copy(col_hbm.at[pl.ds(H + base, _EC_C)], ich)
            cp1 = pltpu.async_copy(tab_hbm.at[irl], brl, s1)
            cp2 = pltpu.async_copy(tab_hbm.at[icl], bcl, s2)
            cp3 = pltpu.async_copy(tab_hbm.at[irh], brh, s3)
            cp4 = pltpu.async_copy(tab_hbm.at[ich], bch, s4)
            cp1.wait()
            cp2.wait()

            def addlo(j, _):
                for s in range(4):
                    ob[j, pl.ds(s * 16, 16)] = (
                        brl[j, pl.ds(s * 16, 16)] + bcl[j, pl.ds(64 + s * 16, 16)])
                return 0

            lax.fori_loop(0, _EC_C, addlo, 0, unroll=2)
            cp3.wait()
            cp4.wait()

            def addhi(j, _):
                for s in range(4):
                    ob[j, pl.ds(64 + s * 16, 16)] = (
                        brh[j, pl.ds(s * 16, 16)] + bch[j, pl.ds(64 + s * 16, 16)])
                return 0

            lax.fori_loop(0, _EC_C, addhi, 0, unroll=2)
            pltpu.sync_copy(ob, out_hbm.at[pl.ds(base, _EC_C)])
        return 0

    lax.fori_loop(0, (_EC_NCHUNK + 31) // 32, step, 0)


def _ec_gather(tab, row, col):
    return pl.kernel(
        _ec_body,
        out_type=jax.ShapeDtypeStruct((H, 128), jnp.float32),
        mesh=plsc.VectorSubcoreMesh(core_axis_name="c", subcore_axis_name="s"),
        scratch_types=[
            pltpu.VMEM((_EC_C,), jnp.int32),
            pltpu.VMEM((_EC_C,), jnp.int32),
            pltpu.VMEM((_EC_C,), jnp.int32),
            pltpu.VMEM((_EC_C,), jnp.int32),
            pltpu.VMEM((_EC_C, 128), jnp.float32),
            pltpu.VMEM((_EC_C, 128), jnp.float32),
            pltpu.VMEM((_EC_C, 128), jnp.float32),
            pltpu.VMEM((_EC_C, 128), jnp.float32),
            pltpu.VMEM((_EC_C, 128), jnp.float32),
            pltpu.SemaphoreType.DMA,
            pltpu.SemaphoreType.DMA,
            pltpu.SemaphoreType.DMA,
            pltpu.SemaphoreType.DMA,
        ],
    )(tab, row, col)


_SS_C = 128
_SS_H = 25016


def _segsum_body(vals_hbm, idx_hbm, out_hbm, vbuf, idxv, idxh, tbl, zb):
    cid = lax.axis_index("c")
    sid = lax.axis_index("s")
    nchunk = NNZ // _SS_C

    def zrow(j, _):
        for s in range(2):
            zb[j, pl.ds(s * 16, 16)] = jnp.zeros((16,), jnp.float32)
        return 0

    lax.fori_loop(0, 256, zrow, 0)

    nz = (2 * _SS_H) // 256

    def zcp(j, _):
        @pl.when(j * 16 + sid < nz)
        def _():
            pltpu.sync_copy(zb, tbl.at[pl.ds((j * 16 + sid) * 256, 256)])
        return 0

    lax.fori_loop(0, (nz + 15) // 16, zcp, 0)

    @pl.when(sid == 0)
    def _():
        pltpu.sync_copy(zb.at[pl.ds(0, 2 * _SS_H - nz * 256)],
                        tbl.at[pl.ds(nz * 256, 2 * _SS_H - nz * 256)])

    plsc.subcore_barrier()

    lane = lax.iota(jnp.int32, 16)

    def step(it, _):
        chunk = it * 16 + sid

        @pl.when(chunk < nchunk)
        def _():
            base = chunk * _SS_C
            pltpu.sync_copy(idx_hbm.at[pl.ds(base, _SS_C)], idxv)
            pltpu.sync_copy(vals_hbm.at[pl.ds(base * 2, 2 * _SS_C)], vbuf)

            def mkidx(j, _):
                v = idxv[pl.ds(j * 16, 16)]
                loc = v - cid * 25000
                oob = (loc < 0) | (loc >= 25000)
                loc = jnp.where(oob, 25000 + ((lane + j) & 15), loc)
                plsc.store_scatter(idxh, [2 * lane + 32 * j], 2 * loc)
                plsc.store_scatter(idxh, [2 * lane + 32 * j + 1], 2 * loc + 1)
                return 0

            lax.fori_loop(0, _SS_C // 16, mkidx, 0)
            pltpu.sync_copy(vbuf, tbl.at[idxh], add=True)
        return 0

    lax.fori_loop(0, (nchunk + 15) // 16, step, 0)
    plsc.subcore_barrier()
    pltpu.sync_copy(tbl.at[pl.ds(sid * 3125, 3125)],
                    out_hbm.at[pl.ds(cid * 50000 + sid * 3125, 3125)])


def _segsum(vals_p, idx_perm):
    v2 = jnp.reshape(vals_p, (2 * NNZ, 32))
    out = pl.kernel(
        _segsum_body,
        out_type=jax.ShapeDtypeStruct((2 * N, 32), jnp.float32),
        mesh=plsc.VectorSubcoreMesh(core_axis_name="c", subcore_axis_name="s"),
        scratch_types=[
            pltpu.VMEM((2 * _SS_C, 32), jnp.float32),
            pltpu.VMEM((_SS_C,), jnp.int32),
            pltpu.VMEM((2 * _SS_C,), jnp.int32),
            pltpu.VMEM_SHARED((2 * _SS_H, 32), jnp.float32),
            pltpu.VMEM((256, 32), jnp.float32),
        ],
        compiler_params=pltpu.CompilerParams(use_tc_tiling_on_sc=False,
                                             needs_layout_passes=False),
    )(v2, idx_perm)
    return jnp.reshape(out, (N, 64))


def _pairperm(x):
    return jnp.stack([x[:H], x[H:]], axis=1).reshape(-1)


def kernel(data_values, data_indices, idx_identity, idx_transpose, W_in, b_in,
           Ws, bs, W_pool, b_pool):
    d = data_values[:, 0]
    row = data_indices[0]
    col = data_indices[1]
    ones = jnp.ones((NNZ,), jnp.float32)

    rs = jax.ops.segment_sum(d, row, num_segments=N)
    rc = jax.ops.segment_sum(ones, row, num_segments=N)
    cs = jax.ops.segment_sum(d, col, num_segments=N)
    cc = jax.ops.segment_sum(ones, col, num_segments=N)
    rmd = rs / jnp.maximum(rc, 1.0)
    cmd = cs / jnp.maximum(cc, 1.0)
    dt = d[idx_transpose]
    di = d[idx_identity]

    w = W_in[0]
    b = b_in
    W0, W1, W2, W3, W4, W5 = [Ws[0, i] for i in range(6)]
    gm = jnp.sum(rs) / NNZ
    k = gm * (w @ W3) + b @ (W0 + W3 + W4 + W5) + bs[0]
    V8 = jnp.stack([w @ W0, w @ W4, w @ W1, w @ W2, w @ W5, b @ W1, b @ W2, k],
                   axis=0)

    feats = [d, dt, rmd[row], cmd[col], di[row], ones, ones, ones]
    f8cm = jnp.stack(feats, axis=0)

    G = _gram(f8cm)
    mA = G[7, :] / NNZ
    mu1 = mA @ V8
    Eo2 = jnp.einsum('ic,ij,jc->c', V8, G / NNZ, V8)
    var1 = Eo2 - mu1 * mu1
    sig1 = jnp.sqrt(var1 + EPS)
    V1 = (V8 - jnp.eye(8, dtype=jnp.float32)[:, 7:8] * mu1[None, :]) / sig1[None, :]

    vals1p = _vals1(f8cm, V1)
    rowp = _pairperm(row)
    colp = _pairperm(col)
    rowsum2 = _segsum(vals1p, rowp)
    colsum2 = _segsum(vals1p, colp)

    ft8cm = jnp.stack([dt] + [jnp.take(f, idx_transpose) for f in feats[1:5]]
                      + [ones, ones, ones], axis=0)
    onn = jnp.ones((N,), jnp.float32)
    fi8rm = jnp.stack([jnp.take(f, idx_identity) for f in feats[:5]]
                      + [onn, onn, onn], axis=1)

    U0, U1, U2, U3, U4, U5 = [Ws[1, i] for i in range(6)]
    tab, gsum = _tabs(rowsum2, colsum2, rc[:, None], cc[:, None], fi8rm,
                      V1, U1, U2, U5)
    g2 = gsum[0] / NNZ
    k2 = (g2 @ U3 + bs[1])[None, :]

    ecp = _ec_gather(tab, row, col)

    out2p, s12 = _out2(f8cm, ft8cm, ecp, V1, U0, U4, k2)
    mu2 = (s12[0, :64] + s12[0, 64:]) / NNZ
    e2 = (s12[1, :64] + s12[1, 64:]) / NNZ
    var2 = e2 - mu2 * mu2
    ms = jnp.stack([mu2, 1.0 / jnp.sqrt(var2 + EPS)], axis=0)
    musig128 = jnp.concatenate([ms, ms], axis=1)
    vals2p = _vals2(out2p, musig128)
    pooled = _segsum(vals2p, rowp)
    return _emb(pooled, rc[:, None], W_pool, b_pool)
